# baseline (speedup 1.0000x reference)
import functools

import jax
import jax.numpy as jnp
from jax import lax
from jax.experimental import pallas as pl
from jax.experimental.pallas import tpu as pltpu
from jax.experimental.pallas import tpu_sc as plsc

_N = 128
_M = 100000
_T = 16384
_MT = 2048
_GRID = -(-_M // _MT)

_NW = 32
_BPW = _T // _NW
_NCH = _BPW // 128
_NV = _N // 16


def _tc_body(a_ref, at_ref, lse_ref, acc_ref):
    i = pl.program_id(0)
    x = a_ref[...]
    col = i * _MT + lax.broadcasted_iota(jnp.int32, x.shape, 1)
    e = jnp.where(col < _M, jnp.exp(x), 0.0)
    partial = jnp.sum(e, axis=1, keepdims=True)

    @pl.when(i == 0)
    def _init():
        acc_ref[...] = jnp.zeros_like(acc_ref)

    acc_ref[...] += partial
    at_ref[...] = x.T

    @pl.when(i == _GRID - 1)
    def _fin():
        lse_ref[...] = jnp.log(acc_ref[...])


def _tc_pass(a):
    return pl.pallas_call(
        _tc_body,
        grid=(_GRID,),
        in_specs=[pl.BlockSpec((_N, _MT), lambda i: (0, i))],
        out_specs=[
            pl.BlockSpec((_MT, _N), lambda i: (i, 0)),
            pl.BlockSpec((_N, 1), lambda i: (0, 0)),
        ],
        out_shape=[
            jax.ShapeDtypeStruct((_M, _N), jnp.float32),
            jax.ShapeDtypeStruct((_N, 1), jnp.float32),
        ],
        scratch_shapes=[pltpu.VMEM((_N, 1), jnp.float32)],
    )(a)


@functools.cache
def _make_sc_gather():
    mesh = plsc.VectorSubcoreMesh(core_axis_name="c", subcore_axis_name="s")
    return pl.kernel(
        _sc_gather_body,
        mesh=mesh,
        out_type=jax.ShapeDtypeStruct((_T, _N), jnp.float32),
        scratch_types=[
            pltpu.VMEM((_NCH, 128), jnp.int32),
            pltpu.VMEM((_BPW, _N), jnp.float32),
            pltpu.VMEM((_N,), jnp.float32),
            pltpu.SemaphoreType.DMA,
        ],
        compiler_params=pltpu.CompilerParams(use_tc_tiling_on_sc=True),
    )


def _sc_gather_body(at_hbm, obs_hbm, lse_hbm, out_hbm, idx_v, rows_v, lse_v, sem):
    wid = lax.axis_index("s") * 2 + lax.axis_index("c")
    base = wid * _BPW
    pltpu.sync_copy(obs_hbm.at[pl.ds(wid * _NCH, _NCH)], idx_v)
    pltpu.sync_copy(lse_hbm, lse_v)
    copies = [
        pltpu.async_copy(
            at_hbm.at[idx_v.at[j]], rows_v.at[pl.ds(j * 128, 128)], sem
        )
        for j in range(_NCH)
    ]
    for c in copies:
        c.wait()
    lvs = [lse_v[pl.ds(16 * j, 16)] for j in range(_NV)]

    def body(i, carry):
        for j in range(_NV):
            sl = pl.ds(16 * j, 16)
            rows_v[i, sl] = rows_v[i, sl] - lvs[j]
        return carry

    lax.fori_loop(0, _BPW, body, 0)
    pltpu.sync_copy(rows_v, out_hbm.at[pl.ds(base, _BPW)])


def kernel(obervation_raw, unnormalized_emission_matrix):
    obs2 = obervation_raw.astype(jnp.int32).reshape(_T // 128, 128)
    at, lse2 = _tc_pass(unnormalized_emission_matrix)
    return _make_sc_gather()(at, obs2, lse2.reshape(_N))


# speedup vs baseline: 2.4106x; 2.4106x over previous
import functools

import jax
import jax.numpy as jnp
from jax import lax
from jax.experimental import pallas as pl
from jax.experimental.pallas import tpu as pltpu
from jax.experimental.pallas import tpu_sc as plsc

_N = 128
_M = 100000
_T = 16384
_MT = 4096
_GRID = -(-_M // _MT)

_NW = 32
_BPW = _T // _NW
_NCH = _BPW // 128
_NV = _N // 16


def _lse_body(at_ref, lse_ref, acc_ref):
    i = pl.program_id(0)
    x = at_ref[...]
    row = i * _MT + lax.broadcasted_iota(jnp.int32, x.shape, 0)
    e = jnp.where(row < _M, jnp.exp(x), 0.0)
    partial = jnp.sum(e, axis=0, keepdims=True)

    @pl.when(i == 0)
    def _init():
        acc_ref[...] = jnp.zeros_like(acc_ref)

    acc_ref[...] += partial

    @pl.when(i == _GRID - 1)
    def _fin():
        lse_ref[...] = jnp.log(acc_ref[...])


def _lse_pass(at):
    return pl.pallas_call(
        _lse_body,
        grid=(_GRID,),
        in_specs=[pl.BlockSpec((_MT, _N), lambda i: (i, 0))],
        out_specs=pl.BlockSpec((1, _N), lambda i: (0, 0)),
        out_shape=jax.ShapeDtypeStruct((1, _N), jnp.float32),
        scratch_shapes=[pltpu.VMEM((1, _N), jnp.float32)],
    )(at)


@functools.cache
def _make_sc_gather():
    mesh = plsc.VectorSubcoreMesh(core_axis_name="c", subcore_axis_name="s")
    return pl.kernel(
        _sc_gather_body,
        mesh=mesh,
        out_type=jax.ShapeDtypeStruct((_T, _N), jnp.float32),
        scratch_types=[
            pltpu.VMEM((_NCH, 128), jnp.int32),
            pltpu.VMEM((_BPW, _N), jnp.float32),
            pltpu.VMEM((_N,), jnp.float32),
            pltpu.SemaphoreType.DMA,
        ],
        compiler_params=pltpu.CompilerParams(use_tc_tiling_on_sc=True),
    )


def _sc_gather_body(at_hbm, obs_hbm, lse_hbm, out_hbm, idx_v, rows_v, lse_v, sem):
    wid = lax.axis_index("s") * 2 + lax.axis_index("c")
    base = wid * _BPW
    pltpu.sync_copy(obs_hbm.at[pl.ds(wid * _NCH, _NCH)], idx_v)
    pltpu.sync_copy(lse_hbm, lse_v)
    copies = [
        pltpu.async_copy(
            at_hbm.at[idx_v.at[j]], rows_v.at[pl.ds(j * 128, 128)], sem
        )
        for j in range(_NCH)
    ]
    for c in copies:
        c.wait()
    lvs = [lse_v[pl.ds(16 * j, 16)] for j in range(_NV)]

    def body(i, carry):
        for j in range(_NV):
            sl = pl.ds(16 * j, 16)
            rows_v[i, sl] = rows_v[i, sl] - lvs[j]
        return carry

    lax.fori_loop(0, _BPW, body, 0)
    pltpu.sync_copy(rows_v, out_hbm.at[pl.ds(base, _BPW)])


def kernel(obervation_raw, unnormalized_emission_matrix):
    obs2 = obervation_raw.astype(jnp.int32).reshape(_T // 128, 128)
    at = unnormalized_emission_matrix.T
    lse2 = _lse_pass(at)
    return _make_sc_gather()(at, obs2, lse2.reshape(_N))


# speedup vs baseline: 2.4165x; 1.0024x over previous
import functools

import jax
import jax.numpy as jnp
from jax import lax
from jax.experimental import pallas as pl
from jax.experimental.pallas import tpu as pltpu
from jax.experimental.pallas import tpu_sc as plsc

_N = 128
_M = 100000
_T = 16384
_MT = 4000
_GRID = _M // _MT

_NW = 32
_BPW = _T // _NW
_NCH = _BPW // 128
_NV = _N // 16


def _lse_body(at_ref, lse_ref, acc_ref):
    i = pl.program_id(0)
    partial = jnp.sum(jnp.exp(at_ref[...]), axis=0, keepdims=True)

    @pl.when(i == 0)
    def _init():
        acc_ref[...] = jnp.zeros_like(acc_ref)

    acc_ref[...] += partial

    @pl.when(i == _GRID - 1)
    def _fin():
        lse_ref[...] = jnp.log(acc_ref[...])


def _lse_pass(at):
    return pl.pallas_call(
        _lse_body,
        grid=(_GRID,),
        in_specs=[pl.BlockSpec((_MT, _N), lambda i: (i, 0))],
        out_specs=pl.BlockSpec((1, _N), lambda i: (0, 0)),
        out_shape=jax.ShapeDtypeStruct((1, _N), jnp.float32),
        scratch_shapes=[pltpu.VMEM((1, _N), jnp.float32)],
    )(at)


@functools.cache
def _make_sc_gather():
    mesh = plsc.VectorSubcoreMesh(core_axis_name="c", subcore_axis_name="s")
    return pl.kernel(
        _sc_gather_body,
        mesh=mesh,
        out_type=jax.ShapeDtypeStruct((_T, _N), jnp.float32),
        scratch_types=[
            pltpu.VMEM((_BPW,), jnp.int32),
            pltpu.VMEM((_BPW, _N), jnp.float32),
            pltpu.VMEM((1, _N), jnp.float32),
            pltpu.SemaphoreType.DMA,
        ],
        compiler_params=pltpu.CompilerParams(use_tc_tiling_on_sc=True),
    )


def _sc_gather_body(at_hbm, obs_hbm, lse_hbm, out_hbm, idx_v, rows_v, lse_v, sem):
    wid = lax.axis_index("s") * 2 + lax.axis_index("c")
    base = wid * _BPW
    pltpu.sync_copy(obs_hbm.at[pl.ds(base, _BPW)], idx_v)
    pltpu.sync_copy(lse_hbm, lse_v)
    copies = [
        pltpu.async_copy(
            at_hbm.at[idx_v.at[pl.ds(j * 128, 128)]],
            rows_v.at[pl.ds(j * 128, 128)],
            sem,
        )
        for j in range(_NCH)
    ]
    for c in copies:
        c.wait()
    lvs = [lse_v[0, pl.ds(16 * j, 16)] for j in range(_NV)]

    def body(i, carry):
        for j in range(_NV):
            sl = pl.ds(16 * j, 16)
            rows_v[i, sl] = rows_v[i, sl] - lvs[j]
        return carry

    lax.fori_loop(0, _BPW, body, 0)
    pltpu.sync_copy(rows_v, out_hbm.at[pl.ds(base, _BPW)])


def kernel(obervation_raw, unnormalized_emission_matrix):
    obs = obervation_raw.astype(jnp.int32)
    at = unnormalized_emission_matrix.T
    lse2 = _lse_pass(at)
    return _make_sc_gather()(at, obs, lse2)


# speedup vs baseline: 2.4902x; 1.0305x over previous
import functools

import jax
import jax.numpy as jnp
from jax import lax
from jax.experimental import pallas as pl
from jax.experimental.pallas import tpu as pltpu
from jax.experimental.pallas import tpu_sc as plsc

_N = 128
_M = 100000
_T = 16384
_MT = 4000
_GRID = _M // _MT

_NW = 32
_BPW = _T // _NW
_NCH = _BPW // 128
_NV = _N // 16


def _lse_body(at_ref, lse_ref, acc_ref):
    i = pl.program_id(0)
    partial = jnp.sum(jnp.exp(at_ref[...]).reshape(_MT // 32, 32, _N), axis=0)

    @pl.when(i == 0)
    def _init():
        acc_ref[...] = jnp.zeros_like(acc_ref)

    acc_ref[...] += partial
    lse_ref[...] = partial[:8][None]

    @pl.when(i == _GRID - 1)
    def _fin():
        lse = jnp.log(jnp.sum(acc_ref[...], axis=0, keepdims=True))
        lse_ref[...] = jnp.broadcast_to(lse, (1, 8, _N))


def _lse_pass(at):
    return pl.pallas_call(
        _lse_body,
        grid=(_GRID,),
        in_specs=[pl.BlockSpec((_MT, _N), lambda i: (i, 0))],
        out_specs=pl.BlockSpec((1, 8, _N), lambda i: (i, 0, 0)),
        out_shape=jax.ShapeDtypeStruct((_GRID, 8, _N), jnp.float32),
        scratch_shapes=[pltpu.VMEM((32, _N), jnp.float32)],
    )(at)


@functools.cache
def _make_sc_gather():
    mesh = plsc.VectorSubcoreMesh(core_axis_name="c", subcore_axis_name="s")
    return pl.kernel(
        _sc_gather_body,
        mesh=mesh,
        out_type=jax.ShapeDtypeStruct((_T, _N), jnp.float32),
        scratch_types=[
            pltpu.VMEM((_BPW,), jnp.int32),
            pltpu.VMEM((_BPW, _N), jnp.float32),
            pltpu.VMEM((1, _N), jnp.float32),
            pltpu.SemaphoreType.DMA,
        ],
        compiler_params=pltpu.CompilerParams(use_tc_tiling_on_sc=True),
    )


def _sc_gather_body(at_hbm, obs_hbm, lse_hbm, out_hbm, idx_v, rows_v, lse_v, sem):
    wid = lax.axis_index("s") * 2 + lax.axis_index("c")
    base = wid * _BPW
    pltpu.sync_copy(obs_hbm.at[pl.ds(base, _BPW)], idx_v)
    pltpu.sync_copy(lse_hbm, lse_v)
    copies = [
        pltpu.async_copy(
            at_hbm.at[idx_v.at[pl.ds(j * 128, 128)]],
            rows_v.at[pl.ds(j * 128, 128)],
            sem,
        )
        for j in range(_NCH)
    ]
    for c in copies:
        c.wait()
    lvs = [lse_v[0, pl.ds(16 * j, 16)] for j in range(_NV)]

    def body(i, carry):
        for j in range(_NV):
            sl = pl.ds(16 * j, 16)
            rows_v[i, sl] = rows_v[i, sl] - lvs[j]
        return carry

    lax.fori_loop(0, _BPW, body, 0)
    pltpu.sync_copy(rows_v, out_hbm.at[pl.ds(base, _BPW)])


def kernel(obervation_raw, unnormalized_emission_matrix):
    obs = obervation_raw.astype(jnp.int32)
    at = unnormalized_emission_matrix.T
    lse2 = _lse_pass(at)[_GRID - 1, :1]
    return _make_sc_gather()(at, obs, lse2)


# speedup vs baseline: 3.0595x; 1.2286x over previous
import functools

import jax
import jax.numpy as jnp
from jax import lax
from jax.experimental import pallas as pl
from jax.experimental.pallas import tpu as pltpu
from jax.experimental.pallas import tpu_sc as plsc

_N = 128
_M = 100000
_T = 16384
_MT = 4000
_GRID = _M // _MT

_NW = 32
_BPW = _T // _NW
_NCH = _BPW // 128
_NV = _N // 16


_CH = 4000
_NCHUNK = _M // _CH
_NBUF = 5


def _lse_body(at_hbm, lse_ref, bufs, sems):
    def _start(c, s):
        pltpu.make_async_copy(
            at_hbm.at[pl.ds(c * _CH, _CH)], bufs.at[s], sems.at[s]
        ).start()

    for s in range(_NBUF):
        _start(s, s)

    def step(c, acc):
        s = c % _NBUF
        pltpu.make_async_copy(
            at_hbm.at[pl.ds(c * _CH, _CH)], bufs.at[s], sems.at[s]
        ).wait()
        acc = acc + jnp.sum(
            jnp.exp(bufs[s]).reshape(_CH // 32, 32, _N), axis=0
        )

        @pl.when(c + _NBUF < _NCHUNK)
        def _next():
            _start(c + _NBUF, s)

        return acc

    acc = lax.fori_loop(
        0, _NCHUNK, step, jnp.zeros((32, _N), jnp.float32)
    )
    lse_ref[...] = jnp.log(jnp.sum(acc, axis=0, keepdims=True))


def _lse_pass(at):
    return pl.pallas_call(
        _lse_body,
        in_specs=[pl.BlockSpec(memory_space=pl.ANY)],
        out_specs=pl.BlockSpec(memory_space=pltpu.VMEM),
        out_shape=jax.ShapeDtypeStruct((1, _N), jnp.float32),
        scratch_shapes=[
            pltpu.VMEM((_NBUF, _CH, _N), jnp.float32),
            pltpu.SemaphoreType.DMA((_NBUF,)),
        ],
    )(at)


@functools.cache
def _make_sc_gather():
    mesh = plsc.VectorSubcoreMesh(core_axis_name="c", subcore_axis_name="s")
    return pl.kernel(
        _sc_gather_body,
        mesh=mesh,
        out_type=jax.ShapeDtypeStruct((_T, _N), jnp.float32),
        scratch_types=[
            pltpu.VMEM((_BPW,), jnp.int32),
            pltpu.VMEM((_BPW, _N), jnp.float32),
            pltpu.VMEM((1, _N), jnp.float32),
            pltpu.SemaphoreType.DMA,
        ],
        compiler_params=pltpu.CompilerParams(use_tc_tiling_on_sc=True),
    )


def _sc_gather_body(at_hbm, obs_hbm, lse_hbm, out_hbm, idx_v, rows_v, lse_v, sem):
    wid = lax.axis_index("s") * 2 + lax.axis_index("c")
    base = wid * _BPW
    pltpu.sync_copy(obs_hbm.at[pl.ds(base, _BPW)], idx_v)
    pltpu.sync_copy(lse_hbm, lse_v)
    copies = [
        pltpu.async_copy(
            at_hbm.at[idx_v.at[pl.ds(j * 128, 128)]],
            rows_v.at[pl.ds(j * 128, 128)],
            sem,
        )
        for j in range(_NCH)
    ]
    for c in copies:
        c.wait()
    lvs = [lse_v[0, pl.ds(16 * j, 16)] for j in range(_NV)]

    def body(i, carry):
        for j in range(_NV):
            sl = pl.ds(16 * j, 16)
            rows_v[i, sl] = rows_v[i, sl] - lvs[j]
        return carry

    lax.fori_loop(0, _BPW, body, 0)
    pltpu.sync_copy(rows_v, out_hbm.at[pl.ds(base, _BPW)])


def kernel(obervation_raw, unnormalized_emission_matrix):
    obs = obervation_raw.astype(jnp.int32)
    at = unnormalized_emission_matrix.T
    lse2 = _lse_pass(at)
    return _make_sc_gather()(at, obs, lse2)


# speedup vs baseline: 3.2486x; 1.0618x over previous
import functools

import jax
import jax.numpy as jnp
from jax import lax
from jax.experimental import pallas as pl
from jax.experimental.pallas import tpu as pltpu
from jax.experimental.pallas import tpu_sc as plsc

_N = 128
_M = 100000
_T = 16384
_MT = 4000
_GRID = _M // _MT

_NW = 32
_BPW = _T // _NW
_NCH = _BPW // 128
_NV = _N // 16


_CH = 4000
_NCHUNK = _M // _CH
_NBUF = 5


def _lse_body(at_hbm, lse_ref, bufs, sems):
    def _start(c, s):
        pltpu.make_async_copy(
            at_hbm.at[pl.ds(c * _CH, _CH)], bufs.at[s], sems.at[s]
        ).start()

    for s in range(_NBUF):
        _start(s, s)

    def step(c, acc):
        s = c % _NBUF
        pltpu.make_async_copy(
            at_hbm.at[pl.ds(c * _CH, _CH)], bufs.at[s], sems.at[s]
        ).wait()
        acc = acc + jnp.sum(
            jnp.exp(bufs[s]).reshape(_CH // 32, 32, _N), axis=0
        )

        @pl.when(c + _NBUF < _NCHUNK)
        def _next():
            _start(c + _NBUF, s)

        return acc

    acc = lax.fori_loop(
        0, _NCHUNK, step, jnp.zeros((32, _N), jnp.float32)
    )
    lse_ref[...] = jnp.log(jnp.sum(acc, axis=0, keepdims=True))


def _lse_pass(at):
    return pl.pallas_call(
        _lse_body,
        in_specs=[pl.BlockSpec(memory_space=pl.ANY)],
        out_specs=pl.BlockSpec(memory_space=pltpu.VMEM),
        out_shape=jax.ShapeDtypeStruct((1, _N), jnp.float32),
        scratch_shapes=[
            pltpu.VMEM((_NBUF, _CH, _N), jnp.float32),
            pltpu.SemaphoreType.DMA((_NBUF,)),
        ],
    )(at)


@functools.cache
def _make_sc_gather():
    mesh = plsc.VectorSubcoreMesh(core_axis_name="c", subcore_axis_name="s")
    return pl.kernel(
        _sc_gather_body,
        mesh=mesh,
        out_type=jax.ShapeDtypeStruct((_T, _N), jnp.float32),
        scratch_types=[
            pltpu.VMEM((_BPW,), jnp.int32),
            pltpu.VMEM((_BPW, _N), jnp.float32),
            pltpu.VMEM((1, _N), jnp.float32),
            pltpu.SemaphoreType.DMA((_NCH,)),
            pltpu.SemaphoreType.DMA,
        ],
        compiler_params=pltpu.CompilerParams(use_tc_tiling_on_sc=True),
    )


def _sc_gather_body(at_hbm, obs_hbm, lse_hbm, out_hbm, idx_v, rows_v, lse_v,
                    gsems, osem):
    wid = lax.axis_index("s") * 2 + lax.axis_index("c")
    base = wid * _BPW
    pltpu.sync_copy(obs_hbm.at[pl.ds(base, _BPW)], idx_v)
    gathers = [
        pltpu.async_copy(
            at_hbm.at[idx_v.at[pl.ds(j * 128, 128)]],
            rows_v.at[pl.ds(j * 128, 128)],
            gsems.at[j],
        )
        for j in range(_NCH)
    ]
    pltpu.sync_copy(lse_hbm, lse_v)
    lvs = [lse_v[0, pl.ds(16 * j, 16)] for j in range(_NV)]

    outs = []
    for j in range(_NCH):
        gathers[j].wait()

        def body(i, carry, _j=j):
            for k in range(_NV):
                sl = pl.ds(16 * k, 16)
                rows_v[_j * 128 + i, sl] = rows_v[_j * 128 + i, sl] - lvs[k]
            return carry

        lax.fori_loop(0, 128, body, 0)
        outs.append(
            pltpu.async_copy(
                rows_v.at[pl.ds(j * 128, 128)],
                out_hbm.at[pl.ds(base + j * 128, 128)],
                osem,
            )
        )
    for c in outs:
        c.wait()


def kernel(obervation_raw, unnormalized_emission_matrix):
    obs = obervation_raw.astype(jnp.int32)
    at = unnormalized_emission_matrix.T
    lse2 = _lse_pass(at)
    return _make_sc_gather()(at, obs, lse2)
